# baseline (speedup 1.0000x reference)
import jax
import jax.numpy as jnp
from jax import lax
from jax.experimental import pallas as pl
from jax.experimental.pallas import tpu as pltpu
from jax.experimental.pallas import tpu_sc as plsc

N_NODES = 100000
N_EDGES = 1600000
D = 32
NS = 16
Q = 25000
Q_PAD = 26624
CHUNK = 512
SUB = 128
NXB = (Q + SUB - 1) // SUB
CW = 16
ZROWS = Q_PAD // NS
ZB = 64
BN = 4000


def _body(x_hbm, t_hbm, idx_hbm, splits_hbm, zeros_hbm, sum_hbm, cnt_hbm,
          acc_sh, cnt_sh,
          t_buf0, t_buf1, idx_raw0, idx_raw1, idx2d, ones_b, zbuf, zcnt,
          splits_v, x_ch, lsem0, lsem1, ssem):
    t_bufs = (t_buf0, t_buf1)
    idx_raws = (idx_raw0, idx_raw1)
    load_sems = (lsem0, lsem1, ssem)
    c0 = lax.axis_index("c")
    s0 = lax.axis_index("s")
    iota = lax.iota(jnp.int32, 16)

    zf = jnp.zeros((16,), jnp.float32)
    of = jnp.ones((16,), jnp.float32)
    pltpu.sync_copy(zeros_hbm, zbuf)
    for r in range(ZB):
        zcnt[r, :] = zf
    for r in range(SUB):
        ones_b[r, :] = of
    pltpu.sync_copy(splits_hbm, splits_v)

    m1 = splits_v[0][0]
    m2 = splits_v[1][0]
    m3 = splits_v[2][0]

    dump = Q + s0

    def one_pass(pp, _):
        qq = 2 * c0 + pp
        e_lo_raw = jnp.where(qq == 0, 0,
                    jnp.where(qq == 1, m1,
                     jnp.where(qq == 2, m2, m3)))
        e_hi_raw = jnp.where(qq == 0, m1,
                    jnp.where(qq == 1, m2,
                     jnp.where(qq == 2, m3, N_EDGES)))
        lo8 = (e_lo_raw // 8) * 8
        hi8 = ((e_hi_raw + 7) // 8) * 8
        node_base = qq * Q

        z0 = s0 * ZROWS
        for j in range(ZROWS // ZB):
            pltpu.sync_copy(zbuf, acc_sh.at[pl.ds(z0 + j * ZB, ZB)])
            pltpu.sync_copy(zcnt, cnt_sh.at[pl.ds(z0 + j * ZB, ZB)])
        plsc.subcore_barrier()

        nc_chunks = (hi8 - lo8 + CHUNK - 1) // CHUNK
        my_chunks = jnp.maximum(0, (nc_chunks - s0 + NS - 1) // NS)

        def chunk_base(j):
            p = lo8 + (s0 + j * NS) * CHUNK
            base = jnp.maximum(0, jnp.minimum(p, N_EDGES - CHUNK))
            return p, pl.multiple_of(base, 8)

        def issue_load(j, b):
            _, base = chunk_base(j)
            pltpu.async_copy(t_hbm.at[pl.ds(base, CHUNK)], t_bufs[b],
                             load_sems[b])
            pltpu.async_copy(idx_hbm.at[pl.ds(base, CHUNK)], idx_raws[b],
                             load_sems[b])

        def wait_load(b):
            pltpu.make_async_copy(t_hbm.at[pl.ds(0, CHUNK)], t_bufs[b],
                                  load_sems[b]).wait()
            pltpu.make_async_copy(idx_hbm.at[pl.ds(0, CHUNK)], idx_raws[b],
                                  load_sems[b]).wait()

        issue_load(0, 0)
        n_pairs = (my_chunks + 1) // 2

        def do_pair(i2, _):
            for b in range(2):
                j = 2 * i2 + b
                p, base = chunk_base(j)
                w_hi = jnp.minimum(p + CHUNK, hi8)
                wait_load(b)
                issue_load(j + 1, 1 - b)

                @pl.when(j < my_chunks)
                def _():
                    for v in range(CHUNK // 16):
                        vec = idx_raws[b][pl.ds(v * 16, 16)]
                        g = base + (v * 16) + iota
                        local = vec - node_base
                        keep = ((g >= p) & (g < w_hi)
                                & (local >= 0) & (local < Q))
                        lidx = jnp.where(keep, local, dump)
                        idx2d[v // 8, pl.ds((v % 8) * 16, 16)] = lidx
                    for j2 in range(CHUNK // SUB):
                        pltpu.sync_copy(t_bufs[b].at[pl.ds(j2 * SUB, SUB)],
                                        acc_sh.at[idx2d.at[j2]], add=True)
                        pltpu.sync_copy(ones_b, cnt_sh.at[idx2d.at[j2]],
                                        add=True)
            return 0

        lax.fori_loop(0, n_pairs, do_pair, 0)
        wait_load(0)
        plsc.subcore_barrier()

        def do_xadd(i, _):
            kf = s0 + i * NS
            lnb_c = jnp.minimum(kf * SUB, Q - SUB)
            gnb_c = node_base + lnb_c

            @pl.when(kf < NXB)
            def _():
                pltpu.sync_copy(x_hbm.at[pl.ds(gnb_c, SUB)], x_ch)
                blk_lo = kf * SUB
                blk_hi = jnp.minimum(blk_lo + SUB, Q)
                for v in range(SUB // 16):
                    loc = lnb_c + v * 16 + iota
                    keep = (loc >= blk_lo) & (loc < blk_hi)
                    idx2d[0, pl.ds(v * 16, 16)] = jnp.where(keep, loc, dump)
                pltpu.sync_copy(x_ch, acc_sh.at[idx2d.at[0]], add=True)
            return 0

        lax.fori_loop(0, (NXB + NS - 1) // NS, do_xadd, 0)
        plsc.subcore_barrier()

        def do_block(i, _):
            kf = s0 + i * NS

            @pl.when(kf < NXB)
            def _():
                lnb_c = jnp.minimum(kf * SUB, Q - SUB)
                gnb_c = node_base + lnb_c
                pltpu.sync_copy(acc_sh.at[pl.ds(lnb_c, SUB)],
                                sum_hbm.at[pl.ds(gnb_c, SUB)])
                pltpu.sync_copy(cnt_sh.at[pl.ds(lnb_c, SUB)],
                                cnt_hbm.at[pl.ds(gnb_c, SUB)])
            return 0

        lax.fori_loop(0, (NXB + NS - 1) // NS, do_block, 0)
        plsc.subcore_barrier()
        return 0

    lax.fori_loop(0, 2, one_pass, 0)


def _scale_body(s_ref, c_ref, o_ref):
    c = c_ref[...][:, 0]
    recip = 1.0 / (1.0 + c)
    o_ref[...] = s_ref[...] * recip[:, None, None]


@jax.jit
def _run(x, t, idx32, splits, zeros):
    mesh = plsc.VectorSubcoreMesh(core_axis_name="c", subcore_axis_name="s")
    f = pl.kernel(
        _body,
        out_type=(
            jax.ShapeDtypeStruct((N_NODES, 4, 8), jnp.float32),
            jax.ShapeDtypeStruct((N_NODES, CW), jnp.float32),
        ),
        mesh=mesh,
        scratch_types=[
            pltpu.VMEM_SHARED((Q_PAD, 4, 8), jnp.float32),
            pltpu.VMEM_SHARED((Q_PAD, CW), jnp.float32),
            pltpu.VMEM((CHUNK, 4, 8), jnp.float32),
            pltpu.VMEM((CHUNK, 4, 8), jnp.float32),
            pltpu.VMEM((CHUNK,), jnp.int32),
            pltpu.VMEM((CHUNK,), jnp.int32),
            pltpu.VMEM((CHUNK // SUB, SUB), jnp.int32),
            pltpu.VMEM((SUB, CW), jnp.float32),
            pltpu.VMEM((ZB, 4, 8), jnp.float32),
            pltpu.VMEM((ZB, CW), jnp.float32),
            pltpu.VMEM((3, 16), jnp.int32),
            pltpu.VMEM((SUB, 4, 8), jnp.float32),
            pltpu.SemaphoreType.DMA,
            pltpu.SemaphoreType.DMA,
            pltpu.SemaphoreType.DMA,
        ],
        compiler_params=pltpu.CompilerParams(use_tc_tiling_on_sc=False),
        name="seg_mean_reduce_sc",
    )
    sums, cnt = f(x, t, idx32, splits, zeros)
    scale = pl.pallas_call(
        _scale_body,
        grid=(N_NODES // BN,),
        in_specs=[
            pl.BlockSpec((BN, 4, 8), lambda i: (i, 0, 0)),
            pl.BlockSpec((BN, CW), lambda i: (i, 0)),
        ],
        out_specs=pl.BlockSpec((BN, 4, 8), lambda i: (i, 0, 0)),
        out_shape=jax.ShapeDtypeStruct((N_NODES, 4, 8), jnp.float32),
        name="seg_mean_scale_tc",
    )
    return scale(sums, cnt)


def kernel(x, t, index):
    idx32 = index.astype(jnp.int32)
    b = jnp.searchsorted(idx32, jnp.array([Q, 2 * Q, 3 * Q], jnp.int32))
    splits = jnp.broadcast_to(b.astype(jnp.int32)[:, None], (3, 16))
    zeros = jnp.zeros((ZB, 4, 8), jnp.float32)
    return _run(x, t, idx32, splits, zeros)


# speedup vs baseline: 3.5894x; 3.5894x over previous
import jax
import jax.numpy as jnp
from jax import lax
from jax.experimental import pallas as pl
from jax.experimental.pallas import tpu as pltpu
from jax.experimental.pallas import tpu_sc as plsc

N_NODES = 100000
N_EDGES = 1600000
D = 32
NS = 16
Q = 25000
Q_PAD = 26624
CHUNK = 512
SUB = 128
NXB = (Q + SUB - 1) // SUB
CW = 16
ZROWS = Q_PAD // NS
ZB = 64
BN = 4000


def _body(t_hbm, idx_hbm, splits_hbm, sum_hbm, cnt_hbm,
          acc_sh, cnt_sh,
          t_buf0, t_buf1, idx_raw0, idx_raw1, idx2d, ones_b, zbuf, zcnt,
          splits_v, lsem0, lsem1, ssem):
    t_bufs = (t_buf0, t_buf1)
    idx_raws = (idx_raw0, idx_raw1)
    load_sems = (lsem0, lsem1, ssem)
    c0 = lax.axis_index("c")
    s0 = lax.axis_index("s")
    iota = lax.iota(jnp.int32, 16)

    zf = jnp.zeros((16,), jnp.float32)
    of = jnp.ones((16,), jnp.float32)
    for r in range(ZB):
        for h in range(D // 16):
            zbuf[r, pl.ds(h * 16, 16)] = zf
        zcnt[r, :] = zf
    for r in range(SUB):
        ones_b[r, :] = of
    pltpu.sync_copy(splits_hbm, splits_v)

    m1 = splits_v[0][0]
    m2 = splits_v[1][0]
    m3 = splits_v[2][0]

    dump = Q + s0

    def one_pass(pp, _):
        qq = 2 * c0 + pp
        e_lo_raw = jnp.where(qq == 0, 0,
                    jnp.where(qq == 1, m1,
                     jnp.where(qq == 2, m2, m3)))
        e_hi_raw = jnp.where(qq == 0, m1,
                    jnp.where(qq == 1, m2,
                     jnp.where(qq == 2, m3, N_EDGES)))
        lo8 = (e_lo_raw // 8) * 8
        hi8 = ((e_hi_raw + 7) // 8) * 8
        node_base = qq * Q

        z0 = s0 * ZROWS
        for j in range(ZROWS // ZB):
            pltpu.sync_copy(zbuf, acc_sh.at[pl.ds(z0 + j * ZB, ZB)])
            pltpu.sync_copy(zcnt, cnt_sh.at[pl.ds(z0 + j * ZB, ZB)])
        plsc.subcore_barrier()

        nc_chunks = (hi8 - lo8 + CHUNK - 1) // CHUNK
        my_chunks = jnp.maximum(0, (nc_chunks - s0 + NS - 1) // NS)

        def chunk_base(j):
            p = lo8 + (s0 + j * NS) * CHUNK
            base = jnp.maximum(0, jnp.minimum(p, N_EDGES - CHUNK))
            return p, pl.multiple_of(base, 8)

        def issue_load(j, b):
            _, base = chunk_base(j)
            pltpu.async_copy(t_hbm.at[pl.ds(base, CHUNK)], t_bufs[b],
                             load_sems[b])
            pltpu.async_copy(idx_hbm.at[pl.ds(base, CHUNK)], idx_raws[b],
                             load_sems[b])

        def wait_load(b):
            pltpu.make_async_copy(t_hbm.at[pl.ds(0, CHUNK)], t_bufs[b],
                                  load_sems[b]).wait()
            pltpu.make_async_copy(idx_hbm.at[pl.ds(0, CHUNK)], idx_raws[b],
                                  load_sems[b]).wait()

        issue_load(0, 0)
        n_pairs = (my_chunks + 1) // 2

        def do_pair(i2, _):
            for b in range(2):
                j = 2 * i2 + b
                p, base = chunk_base(j)
                w_hi = jnp.minimum(p + CHUNK, hi8)
                wait_load(b)
                issue_load(j + 1, 1 - b)

                @pl.when(j < my_chunks)
                def _():
                    for v in range(CHUNK // 16):
                        vec = idx_raws[b][pl.ds(v * 16, 16)]
                        g = base + (v * 16) + iota
                        local = vec - node_base
                        keep = ((g >= p) & (g < w_hi)
                                & (local >= 0) & (local < Q))
                        lidx = jnp.where(keep, local, dump)
                        idx2d[v // 8, pl.ds((v % 8) * 16, 16)] = lidx
                    for j2 in range(CHUNK // SUB):
                        pltpu.sync_copy(t_bufs[b].at[pl.ds(j2 * SUB, SUB)],
                                        acc_sh.at[idx2d.at[j2]], add=True)
                        pltpu.sync_copy(ones_b, cnt_sh.at[idx2d.at[j2]],
                                        add=True)
            return 0

        lax.fori_loop(0, n_pairs, do_pair, 0)
        wait_load(0)
        plsc.subcore_barrier()

        def do_block(i, _):
            kf = s0 + i * NS

            @pl.when(kf < NXB)
            def _():
                lnb = jnp.minimum(kf * SUB, Q - SUB)
                gnb = node_base + lnb
                pltpu.sync_copy(acc_sh.at[pl.ds(lnb, SUB)],
                                sum_hbm.at[pl.ds(gnb, SUB)])
                pltpu.sync_copy(cnt_sh.at[pl.ds(lnb, SUB)],
                                cnt_hbm.at[pl.ds(gnb, SUB)])
            return 0

        lax.fori_loop(0, (NXB + NS - 1) // NS, do_block, 0)
        plsc.subcore_barrier()
        return 0

    lax.fori_loop(0, 2, one_pass, 0)


def _scale_body(x_ref, s_ref, c_ref, o_ref):
    c = c_ref[...][:, 0]
    recip = 1.0 / (1.0 + c)
    s = s_ref[...].reshape(BN, 4, 8)
    o_ref[...] = (x_ref[...] + s) * recip[:, None, None]


@jax.jit
def _run(x3, t2, idx32, splits):
    mesh = plsc.VectorSubcoreMesh(core_axis_name="c", subcore_axis_name="s")
    f = pl.kernel(
        _body,
        out_type=(
            jax.ShapeDtypeStruct((N_NODES, D), jnp.float32),
            jax.ShapeDtypeStruct((N_NODES, CW), jnp.float32),
        ),
        mesh=mesh,
        scratch_types=[
            pltpu.VMEM_SHARED((Q_PAD, D), jnp.float32),
            pltpu.VMEM_SHARED((Q_PAD, CW), jnp.float32),
            pltpu.VMEM((CHUNK, D), jnp.float32),
            pltpu.VMEM((CHUNK, D), jnp.float32),
            pltpu.VMEM((CHUNK,), jnp.int32),
            pltpu.VMEM((CHUNK,), jnp.int32),
            pltpu.VMEM((CHUNK // SUB, SUB), jnp.int32),
            pltpu.VMEM((SUB, CW), jnp.float32),
            pltpu.VMEM((ZB, D), jnp.float32),
            pltpu.VMEM((ZB, CW), jnp.float32),
            pltpu.VMEM((3, 16), jnp.int32),
            pltpu.SemaphoreType.DMA,
            pltpu.SemaphoreType.DMA,
            pltpu.SemaphoreType.DMA,
        ],
        compiler_params=pltpu.CompilerParams(use_tc_tiling_on_sc=False),
        name="seg_mean_reduce_sc",
    )
    sums, cnt = f(t2, idx32, splits)
    scale = pl.pallas_call(
        _scale_body,
        grid=(N_NODES // BN,),
        in_specs=[
            pl.BlockSpec((BN, 4, 8), lambda i: (i, 0, 0)),
            pl.BlockSpec((BN, D), lambda i: (i, 0)),
            pl.BlockSpec((BN, CW), lambda i: (i, 0)),
        ],
        out_specs=pl.BlockSpec((BN, 4, 8), lambda i: (i, 0, 0)),
        out_shape=jax.ShapeDtypeStruct((N_NODES, 4, 8), jnp.float32),
        name="seg_mean_scale_tc",
    )
    return scale(x3, sums, cnt)


def kernel(x, t, index):
    idx32 = index.astype(jnp.int32)
    b = jnp.searchsorted(idx32, jnp.array([Q, 2 * Q, 3 * Q], jnp.int32))
    splits = jnp.broadcast_to(b.astype(jnp.int32)[:, None], (3, 16))
    t2 = t.reshape(N_EDGES, D)
    return _run(x, t2, idx32, splits)


# speedup vs baseline: 4.4793x; 1.2479x over previous
import jax
import jax.numpy as jnp
from jax import lax
from jax.experimental import pallas as pl
from jax.experimental.pallas import tpu as pltpu
from jax.experimental.pallas import tpu_sc as plsc

N_NODES = 100000
N_EDGES = 1600000
D = 32
NS = 16
Q = 25000
Q_PAD = 26624
CHUNK = 512
SUB = 128
NXB = (Q + SUB - 1) // SUB
CW = 16
ZROWS = Q_PAD // NS
ZB = 64


def _body(x_hbm, t_hbm, idx_hbm, splits_hbm, out_hbm,
          acc_sh, cnt_sh,
          t_buf0, t_buf1, idx_raw0, idx_raw1, idx2d, ones_b, zbuf, zcnt,
          splits_v, lsem0, lsem1, ssem):
    t_bufs = (t_buf0, t_buf1)
    idx_raws = (idx_raw0, idx_raw1)
    load_sems = (lsem0, lsem1, ssem)
    c0 = lax.axis_index("c")
    s0 = lax.axis_index("s")
    iota = lax.iota(jnp.int32, 16)

    zf = jnp.zeros((16,), jnp.float32)
    of = jnp.ones((16,), jnp.float32)
    for r in range(ZB):
        for h in range(D // 16):
            zbuf[r, pl.ds(h * 16, 16)] = zf
        zcnt[r, :] = zf
    for r in range(SUB):
        ones_b[r, :] = of
    pltpu.sync_copy(splits_hbm, splits_v)

    m1 = splits_v[0][0]
    m2 = splits_v[1][0]
    m3 = splits_v[2][0]

    dump = Q + s0

    def one_pass(pp, _):
        qq = 2 * c0 + pp
        e_lo_raw = jnp.where(qq == 0, 0,
                    jnp.where(qq == 1, m1,
                     jnp.where(qq == 2, m2, m3)))
        e_hi_raw = jnp.where(qq == 0, m1,
                    jnp.where(qq == 1, m2,
                     jnp.where(qq == 2, m3, N_EDGES)))
        lo8 = (e_lo_raw // 8) * 8
        hi8 = ((e_hi_raw + 7) // 8) * 8
        node_base = qq * Q

        z0 = s0 * ZROWS
        for j in range(ZROWS // ZB):
            pltpu.sync_copy(zbuf, acc_sh.at[pl.ds(z0 + j * ZB, ZB)])
            pltpu.sync_copy(zcnt, cnt_sh.at[pl.ds(z0 + j * ZB, ZB)])
        plsc.subcore_barrier()

        nc_chunks = (hi8 - lo8 + CHUNK - 1) // CHUNK
        my_chunks = jnp.maximum(0, (nc_chunks - s0 + NS - 1) // NS)

        def chunk_base(j):
            p = lo8 + (s0 + j * NS) * CHUNK
            base = jnp.maximum(0, jnp.minimum(p, N_EDGES - CHUNK))
            return p, pl.multiple_of(base, 8)

        def issue_load(j, b):
            _, base = chunk_base(j)
            pltpu.async_copy(t_hbm.at[pl.ds(base, CHUNK)], t_bufs[b],
                             load_sems[b])
            pltpu.async_copy(idx_hbm.at[pl.ds(base, CHUNK)], idx_raws[b],
                             load_sems[b])

        def wait_load(b):
            pltpu.make_async_copy(t_hbm.at[pl.ds(0, CHUNK)], t_bufs[b],
                                  load_sems[b]).wait()
            pltpu.make_async_copy(idx_hbm.at[pl.ds(0, CHUNK)], idx_raws[b],
                                  load_sems[b]).wait()

        issue_load(0, 0)
        n_pairs = (my_chunks + 1) // 2

        def do_pair(i2, _):
            for b in range(2):
                j = 2 * i2 + b
                p, base = chunk_base(j)
                w_hi = jnp.minimum(p + CHUNK, hi8)
                wait_load(b)
                issue_load(j + 1, 1 - b)

                @pl.when(j < my_chunks)
                def _():
                    for v in range(CHUNK // 16):
                        vec = idx_raws[b][pl.ds(v * 16, 16)]
                        g = base + (v * 16) + iota
                        local = vec - node_base
                        keep = ((g >= p) & (g < w_hi)
                                & (local >= 0) & (local < Q))
                        lidx = jnp.where(keep, local, dump)
                        idx2d[v // 8, pl.ds((v % 8) * 16, 16)] = lidx
                    for j2 in range(CHUNK // SUB):
                        pltpu.sync_copy(t_bufs[b].at[pl.ds(j2 * SUB, SUB)],
                                        acc_sh.at[idx2d.at[j2]], add=True)
                        pltpu.sync_copy(ones_b, cnt_sh.at[idx2d.at[j2]],
                                        add=True)
            return 0

        lax.fori_loop(0, n_pairs, do_pair, 0)
        wait_load(0)
        plsc.subcore_barrier()

        def do_block(i, _):
            kf = s0 + i * NS

            @pl.when(kf < NXB)
            def _():
                lnb = jnp.minimum(kf * SUB, Q - SUB)
                gnb = node_base + lnb
                pltpu.sync_copy(x_hbm.at[pl.ds(gnb, SUB)],
                                t_buf0.at[pl.ds(0, SUB)])
                pltpu.sync_copy(acc_sh.at[pl.ds(lnb, SUB)],
                                t_buf0.at[pl.ds(SUB, SUB)])
                pltpu.sync_copy(cnt_sh.at[pl.ds(lnb, SUB)],
                                t_buf0.at[pl.ds(2 * SUB, SUB), pl.ds(0, CW)])
                for r in range(SUB):
                    recip = 1.0 / (1.0 + t_buf0[2 * SUB + r, pl.ds(0, CW)])
                    for h in range(D // 16):
                        sl = pl.ds(h * 16, 16)
                        t_buf1[r, sl] = ((t_buf0[r, sl]
                                          + t_buf0[SUB + r, sl]) * recip)
                pltpu.sync_copy(t_buf1.at[pl.ds(0, SUB)],
                                out_hbm.at[pl.ds(gnb, SUB)])
            return 0

        lax.fori_loop(0, (NXB + NS - 1) // NS, do_block, 0)
        plsc.subcore_barrier()
        return 0

    lax.fori_loop(0, 2, one_pass, 0)


@jax.jit
def _run(x2, t2, idx32, splits):
    mesh = plsc.VectorSubcoreMesh(core_axis_name="c", subcore_axis_name="s")
    f = pl.kernel(
        _body,
        out_type=jax.ShapeDtypeStruct((N_NODES, D), jnp.float32),
        mesh=mesh,
        scratch_types=[
            pltpu.VMEM_SHARED((Q_PAD, D), jnp.float32),
            pltpu.VMEM_SHARED((Q_PAD, CW), jnp.float32),
            pltpu.VMEM((CHUNK, D), jnp.float32),
            pltpu.VMEM((CHUNK, D), jnp.float32),
            pltpu.VMEM((CHUNK,), jnp.int32),
            pltpu.VMEM((CHUNK,), jnp.int32),
            pltpu.VMEM((CHUNK // SUB, SUB), jnp.int32),
            pltpu.VMEM((SUB, CW), jnp.float32),
            pltpu.VMEM((ZB, D), jnp.float32),
            pltpu.VMEM((ZB, CW), jnp.float32),
            pltpu.VMEM((3, 16), jnp.int32),
            pltpu.SemaphoreType.DMA,
            pltpu.SemaphoreType.DMA,
            pltpu.SemaphoreType.DMA,
        ],
        compiler_params=pltpu.CompilerParams(use_tc_tiling_on_sc=False),
        name="seg_mean_reduce_sc",
    )
    return f(x2, t2, idx32, splits)


def kernel(x, t, index):
    idx32 = index.astype(jnp.int32)
    b = jnp.searchsorted(idx32, jnp.array([Q, 2 * Q, 3 * Q], jnp.int32))
    splits = jnp.broadcast_to(b.astype(jnp.int32)[:, None], (3, 16))
    x2 = x.reshape(N_NODES, D)
    t2 = t.reshape(N_EDGES, D)
    return _run(x2, t2, idx32, splits).reshape(N_NODES, 4, 8)
